# baseline (speedup 1.0000x reference)
import jax
import jax.numpy as jnp
from jax import lax
from jax.experimental import pallas as pl
from jax.experimental.pallas import tpu as pltpu
from jax.experimental.pallas import tpu_sc as plsc

BATCH = 16384
EMBED_DIM = 16
NUM_CORES = 2
NUM_SUBCORES = 16
NUM_WORKERS = NUM_CORES * NUM_SUBCORES
BPW = BATCH // NUM_WORKERS
LANES = 16
NCH = BPW // LANES


def _body(data_hbm, user_hbm, item_hbm, out_hbm,
          data_v, ub0, ub1, ib0, ib1, out_v, sem0, sem1):
    u_bufs = (ub0, ub1)
    i_bufs = (ib0, ib1)
    sems = (sem0, sem1)

    wid = lax.axis_index("s") * NUM_CORES + lax.axis_index("c")
    base = wid * BPW

    pltpu.sync_copy(data_hbm.at[pl.ds(base * 2, BPW * 2)], data_v)

    lanes = lax.iota(jnp.int32, 16)

    def fire(c, par):
        iu = plsc.load_gather(data_v, [2 * (c * LANES + lanes)])
        ii = plsc.load_gather(data_v, [2 * (c * LANES + lanes) + 1])
        for j in range(LANES):
            pltpu.async_copy(
                user_hbm.at[pl.ds(iu[j] * EMBED_DIM, EMBED_DIM)],
                u_bufs[par].at[j], sems[par])
            pltpu.async_copy(
                item_hbm.at[pl.ds(ii[j] * EMBED_DIM, EMBED_DIM)],
                i_bufs[par].at[j], sems[par])

    def drain(par):
        for j in range(LANES):
            pltpu.make_async_copy(
                user_hbm.at[pl.ds(0, EMBED_DIM)],
                u_bufs[par].at[j], sems[par]).wait()
            pltpu.make_async_copy(
                item_hbm.at[pl.ds(0, EMBED_DIM)],
                i_bufs[par].at[j], sems[par]).wait()

    def compute(c, par):
        acc = jnp.zeros((16,), jnp.float32)
        for d in range(EMBED_DIM):
            col = jnp.full((16,), d, jnp.int32)
            acc = acc + (plsc.load_gather(u_bufs[par], [lanes, col]) *
                         plsc.load_gather(i_bufs[par], [lanes, col]))
        out_v[pl.ds(c * LANES, LANES)] = 1.0 / (1.0 + jnp.exp(-acc))

    fire(0, 0)

    def pair(j, _):
        a = 2 * j
        fire(a + 1, 1)
        drain(0)
        compute(a, 0)

        @pl.when(j < NCH // 2 - 1)
        def _():
            fire(a + 2, 0)

        drain(1)
        compute(a + 1, 1)
        return 0

    lax.fori_loop(0, NCH // 2, pair, 0)

    pltpu.sync_copy(out_v, out_hbm.at[pl.ds(base, BPW)])


@jax.jit
def _run(data_flat, user_flat, item_flat):
    mesh = plsc.VectorSubcoreMesh(
        core_axis_name="c", subcore_axis_name="s",
        num_cores=NUM_CORES, num_subcores=NUM_SUBCORES)
    scratch = [
        pltpu.VMEM((BPW * 2,), jnp.int32),
        pltpu.VMEM((LANES, EMBED_DIM), jnp.float32),
        pltpu.VMEM((LANES, EMBED_DIM), jnp.float32),
        pltpu.VMEM((LANES, EMBED_DIM), jnp.float32),
        pltpu.VMEM((LANES, EMBED_DIM), jnp.float32),
        pltpu.VMEM((BPW,), jnp.float32),
        pltpu.SemaphoreType.DMA,
        pltpu.SemaphoreType.DMA,
    ]
    f = pl.kernel(
        _body,
        out_type=jax.ShapeDtypeStruct((BATCH,), jnp.float32),
        mesh=mesh,
        scratch_types=scratch,
        compiler_params=pltpu.CompilerParams(
            needs_layout_passes=False, use_tc_tiling_on_sc=False),
    )
    return f(data_flat, user_flat, item_flat)


def kernel(data, user_embeds, item_embeds):
    data_flat = data.astype(jnp.int32).reshape(-1)
    user_flat = user_embeds.reshape(-1)
    item_flat = item_embeds.reshape(-1)
    return _run(data_flat, user_flat, item_flat)
